# baseline (speedup 1.0000x reference)
import functools

import jax
import jax.numpy as jnp
from jax.experimental import pallas as pl
from jax.experimental.pallas import tpu as pltpu

B = 4096
D = 1024
K = 64
N_SAMPLES = 1000
NSTREAM = 4
GRID = 4
SUB = B // (NSTREAM * GRID)
GROUP = B // NSTREAM


def _mix_kernel(u_ref, cw_ref, mus_ref, lv_ref, *refs):
    x_refs = refs[:NSTREAM]
    elbo_refs = refs[NSTREAM:2 * NSTREAM]
    loss_ref = refs[2 * NSTREAM]
    wiv_s, wmiv_s, c_s, acc_s = refs[2 * NSTREAM + 1:]
    i = pl.program_id(0)

    @pl.when(i == 0)
    def _prologue():
        u = u_ref[:]
        g = -jnp.log(-jnp.log(u)) + cw_ref[:]
        rowmax = jnp.max(g, axis=1, keepdims=True)
        col = jax.lax.broadcasted_iota(jnp.int32, g.shape, 1)
        idx = jnp.where(g == rowmax, col, K)
        amin = jnp.min(idx, axis=1, keepdims=True)
        firsthot = (col == amin).astype(jnp.float32)
        w = jnp.sum(firsthot, axis=0, keepdims=True) / N_SAMPLES

        lv = lv_ref[:]
        iv = jnp.exp(-lv)
        mus = mus_ref[:]
        dot = functools.partial(jax.lax.dot_general,
                                dimension_numbers=(((1,), (0,)), ((), ())),
                                precision=jax.lax.Precision.HIGHEST,
                                preferred_element_type=jnp.float32)
        wiv_s[:] = dot(w, iv)
        wmiv_s[:] = 2.0 * dot(w, mus * iv)
        t = jnp.sum(mus * mus * iv + lv, axis=1, keepdims=True)
        c_s[0, 0] = dot(w, t)[0, 0] + D * jnp.log(2.0 * jnp.pi)
        acc_s[0, 0] = 0.0

    c = c_s[0, 0]
    wiv = wiv_s[:]
    wmiv2 = wmiv_s[:]
    total = 0.0
    for g_idx in range(NSTREAM):
        xb = x_refs[g_idx][:]
        row = jnp.sum(xb * (xb * wiv - wmiv2), axis=1)
        elbo_refs[g_idx][:] = (-0.5 * (row + c)).reshape(SUB, 1)
        total += jnp.sum(row)
    acc_s[0, 0] += total

    @pl.when(i == GRID - 1)
    def _epilogue():
        loss_ref[:] = jnp.full((1, 1), 0.5 * (acc_s[0, 0] / B + c_s[0, 0]),
                               dtype=jnp.float32)


def kernel(x, categorical_weights, mus, log_var):
    key = jax.random.key(42)
    u = jax.random.uniform(key, (N_SAMPLES, K), jnp.float32,
                           minval=jnp.finfo(jnp.float32).tiny, maxval=1.0)
    cw = categorical_weights.reshape(1, K)

    x_specs = [
        pl.BlockSpec((SUB, D), functools.partial(lambda g, i: (g * GRID + i, 0), g))
        for g in range(NSTREAM)
    ]
    outs = pl.pallas_call(
        _mix_kernel,
        grid=(GRID,),
        in_specs=[
            pl.BlockSpec((N_SAMPLES, K), lambda i: (0, 0)),
            pl.BlockSpec((1, K), lambda i: (0, 0)),
            pl.BlockSpec((K, D), lambda i: (0, 0)),
            pl.BlockSpec((K, D), lambda i: (0, 0)),
            *x_specs,
        ],
        out_specs=[
            *[pl.BlockSpec((SUB, 1), lambda i: (i, 0)) for _ in range(NSTREAM)],
            pl.BlockSpec((1, 1), lambda i: (0, 0)),
        ],
        out_shape=[
            *[jax.ShapeDtypeStruct((GROUP, 1), jnp.float32) for _ in range(NSTREAM)],
            jax.ShapeDtypeStruct((1, 1), jnp.float32),
        ],
        scratch_shapes=[
            pltpu.VMEM((1, D), jnp.float32),
            pltpu.VMEM((1, D), jnp.float32),
            pltpu.SMEM((1, 1), jnp.float32),
            pltpu.SMEM((1, 1), jnp.float32),
        ],
    )(u, cw, mus, log_var, *([x] * NSTREAM))

    elbo = jnp.concatenate(outs[:NSTREAM], axis=0)[:, 0]
    return outs[NSTREAM][0, 0], elbo


# speedup vs baseline: 1.3102x; 1.3102x over previous
import functools

import jax
import jax.numpy as jnp
from jax.experimental import pallas as pl
from jax.experimental.pallas import tpu as pltpu

B = 4096
D = 1024
K = 64
N_SAMPLES = 1000
NSTREAM = 1
GRID = 2
SUB = B // (NSTREAM * GRID)
GROUP = B // NSTREAM


def _mix_kernel(u_ref, cw_ref, mus_ref, lv_ref, *refs):
    x_refs = refs[:NSTREAM]
    elbo_refs = refs[NSTREAM:2 * NSTREAM]
    loss_ref = refs[2 * NSTREAM]
    wiv_s, wmiv_s, c_s, acc_s = refs[2 * NSTREAM + 1:]
    i = pl.program_id(0)

    @pl.when(i == 0)
    def _prologue():
        u = u_ref[:]
        g = -jnp.log(-jnp.log(u)) + cw_ref[:]
        rowmax = jnp.max(g, axis=1, keepdims=True)
        col = jax.lax.broadcasted_iota(jnp.int32, g.shape, 1)
        idx = jnp.where(g == rowmax, col, K)
        amin = jnp.min(idx, axis=1, keepdims=True)
        firsthot = (col == amin).astype(jnp.float32)
        w = jnp.sum(firsthot, axis=0, keepdims=True) / N_SAMPLES

        lv = lv_ref[:]
        iv = jnp.exp(-lv)
        mus = mus_ref[:]
        dot = functools.partial(jax.lax.dot_general,
                                dimension_numbers=(((1,), (0,)), ((), ())),
                                precision=jax.lax.Precision.HIGHEST,
                                preferred_element_type=jnp.float32)
        wiv_s[:] = dot(w, iv)
        wmiv_s[:] = 2.0 * dot(w, mus * iv)
        t = jnp.sum(mus * mus * iv + lv, axis=1, keepdims=True)
        c_s[0, 0] = dot(w, t)[0, 0] + D * jnp.log(2.0 * jnp.pi)
        acc_s[0, 0] = 0.0

    c = c_s[0, 0]
    wiv = wiv_s[:]
    wmiv2 = wmiv_s[:]
    total = 0.0
    for g_idx in range(NSTREAM):
        xb = x_refs[g_idx][:]
        row = jnp.sum(xb * (xb * wiv - wmiv2), axis=1)
        elbo_refs[g_idx][:] = (-0.5 * (row + c)).reshape(SUB, 1)
        total += jnp.sum(row)
    acc_s[0, 0] += total

    @pl.when(i == GRID - 1)
    def _epilogue():
        loss_ref[:] = jnp.full((1, 1), 0.5 * (acc_s[0, 0] / B + c_s[0, 0]),
                               dtype=jnp.float32)


def kernel(x, categorical_weights, mus, log_var):
    key = jax.random.key(42)
    u = jax.random.uniform(key, (N_SAMPLES, K), jnp.float32,
                           minval=jnp.finfo(jnp.float32).tiny, maxval=1.0)
    cw = categorical_weights.reshape(1, K)

    x_specs = [
        pl.BlockSpec((SUB, D), functools.partial(lambda g, i: (g * GRID + i, 0), g))
        for g in range(NSTREAM)
    ]
    outs = pl.pallas_call(
        _mix_kernel,
        grid=(GRID,),
        in_specs=[
            pl.BlockSpec((N_SAMPLES, K), lambda i: (0, 0)),
            pl.BlockSpec((1, K), lambda i: (0, 0)),
            pl.BlockSpec((K, D), lambda i: (0, 0)),
            pl.BlockSpec((K, D), lambda i: (0, 0)),
            *x_specs,
        ],
        out_specs=[
            *[pl.BlockSpec((SUB, 1), lambda i: (i, 0)) for _ in range(NSTREAM)],
            pl.BlockSpec((1, 1), lambda i: (0, 0)),
        ],
        out_shape=[
            *[jax.ShapeDtypeStruct((GROUP, 1), jnp.float32) for _ in range(NSTREAM)],
            jax.ShapeDtypeStruct((1, 1), jnp.float32),
        ],
        scratch_shapes=[
            pltpu.VMEM((1, D), jnp.float32),
            pltpu.VMEM((1, D), jnp.float32),
            pltpu.SMEM((1, 1), jnp.float32),
            pltpu.SMEM((1, 1), jnp.float32),
        ],
    )(u, cw, mus, log_var, *([x] * NSTREAM))

    elbo = jnp.concatenate(outs[:NSTREAM], axis=0)[:, 0]
    return outs[NSTREAM][0, 0], elbo
